# baseline (speedup 1.0000x reference)
import functools

import jax
import jax.numpy as jnp
from jax import lax
from jax.experimental import pallas as pl
from jax.experimental.pallas import tpu as pltpu
from jax.experimental.pallas import tpu_sc as plsc

N, E, F, H, ED, L, G = 10000, 320000, 128, 32, 16, 4, 64

TW = 48
N_PAD = 10016
NW = 32
E_PAD = 327680
EPW = E_PAD // NW
CH = 128
SCH = 512
K = SCH // CH
NSCH = EPW // SCH
RPT = N_PAD // 16


@functools.lru_cache(maxsize=1)
def _sc_kernels():
    mesh = plsc.VectorSubcoreMesh(core_axis_name="c", subcore_axis_name="s")
    cparams = pltpu.CompilerParams(use_tc_tiling_on_sc=False)

    @functools.partial(
        pl.kernel,
        mesh=mesh,
        compiler_params=cparams,
        out_type=jax.ShapeDtypeStruct((E_PAD, 128), jnp.float32),
        scratch_types=[
            pltpu.VMEM((K, CH), jnp.int32),
            pltpu.VMEM((K, CH), jnp.int32),
            pltpu.VMEM((K, CH), jnp.int32),
            pltpu.VMEM((K, CH), jnp.int32),
            pltpu.VMEM((SCH, TW), jnp.float32),
            pltpu.VMEM((SCH, TW), jnp.float32),
            pltpu.VMEM((SCH, TW), jnp.float32),
            pltpu.VMEM((SCH, TW), jnp.float32),
            pltpu.VMEM((SCH, ED), jnp.float32),
            pltpu.VMEM((SCH, ED), jnp.float32),
            pltpu.SemaphoreType.DMA,
            pltpu.SemaphoreType.DMA,
        ],
    )
    def sc_gather(tbl, row2, col2, ea, g_out,
                  ir0, ir1, ic0, ic1, br0, br1, bc0, bc1, be0, be1,
                  sem0, sem1):
        cid = lax.axis_index("c")
        sid = lax.axis_index("s")
        wid = sid * 2 + cid
        b0 = wid * EPW
        idx_r, idx_c = (ir0, ir1), (ic0, ic1)
        buf_r, buf_c = (br0, br1), (bc0, bc1)
        buf_e = (be0, be1)
        sems = (sem0, sem1)

        def fire(j, s):
            base = b0 + j * SCH
            pltpu.sync_copy(row2.at[pl.ds(base // CH, K)], idx_r[s])
            pltpu.sync_copy(col2.at[pl.ds(base // CH, K)], idx_c[s])
            pltpu.async_copy(ea.at[pl.ds(base, SCH)], buf_e[s], sems[s])
            for k in range(K):
                pltpu.async_copy(tbl.at[idx_r[s].at[k]],
                                 buf_r[s].at[pl.ds(k * CH, CH)], sems[s])
                pltpu.async_copy(tbl.at[idx_c[s].at[k]],
                                 buf_c[s].at[pl.ds(k * CH, CH)], sems[s])

        def drain_write(j, s):
            base = b0 + j * SCH
            dst_r = g_out.at[pl.ds(base, SCH), pl.ds(0, TW)]
            dst_c = g_out.at[pl.ds(base, SCH), pl.ds(TW, TW)]
            dst_e = g_out.at[pl.ds(base, SCH), pl.ds(2 * TW, ED)]
            pltpu.make_async_copy(dst_r, buf_r[s], sems[s]).wait()
            pltpu.make_async_copy(dst_c, buf_c[s], sems[s]).wait()
            pltpu.make_async_copy(dst_e, buf_e[s], sems[s]).wait()
            pltpu.sync_copy(buf_r[s], dst_r)
            pltpu.sync_copy(buf_c[s], dst_c)
            pltpu.sync_copy(buf_e[s], dst_e)

        fire(0, 0)

        def body(jj, carry):
            for s in (0, 1):
                j = 2 * jj + s

                @pl.when(j + 1 < NSCH)
                def _(j=j, s=s):
                    fire(j + 1, s ^ 1)

                drain_write(j, s)
            return carry

        lax.fori_loop(0, NSCH // 2, body, 0)

    def make_scatter(width):
        @functools.partial(
            pl.kernel,
            mesh=mesh,
            compiler_params=cparams,
            out_type=jax.ShapeDtypeStruct((2, N_PAD, width), jnp.float32),
            scratch_types=[
                pltpu.VMEM((K, CH), jnp.int32),
                pltpu.VMEM((K, CH), jnp.int32),
                pltpu.VMEM((SCH, width), jnp.float32),
                pltpu.VMEM((SCH, width), jnp.float32),
                pltpu.VMEM_SHARED((N_PAD, width), jnp.float32),
                pltpu.SemaphoreType.DMA,
                pltpu.SemaphoreType.DMA,
            ],
        )
        def sc_scatter(s_hbm, row2, z_hbm, out_hbm,
                       i0, i1, bf0, bf1, acc, sem0, sem1):
            cid = lax.axis_index("c")
            sid = lax.axis_index("s")
            r0 = sid * RPT
            b0 = (cid * 16 + sid) * EPW
            idx_v = (i0, i1)
            buf = (bf0, bf1)
            sems = (sem0, sem1)

            def fire(j, s):
                base = b0 + j * SCH
                pltpu.async_copy(row2.at[pl.ds(base // CH, K)], idx_v[s], sems[s])
                pltpu.async_copy(s_hbm.at[pl.ds(base, SCH), pl.ds(0, width)],
                                 buf[s], sems[s])

            def drain_add(j, s):
                base = b0 + j * SCH
                pltpu.make_async_copy(row2.at[pl.ds(base // CH, K)],
                                      idx_v[s], sems[s]).wait()
                pltpu.make_async_copy(s_hbm.at[pl.ds(base, SCH), pl.ds(0, width)],
                                      buf[s], sems[s]).wait()
                for k in range(K):
                    pltpu.sync_copy(buf[s].at[pl.ds(k * CH, CH)],
                                    acc.at[idx_v[s].at[k]], add=True)

            fire(0, 0)
            pltpu.sync_copy(z_hbm.at[pl.ds(r0, RPT)], acc.at[pl.ds(r0, RPT)])
            plsc.subcore_barrier()

            def body(jj, carry):
                for s in (0, 1):
                    j = 2 * jj + s

                    @pl.when(j + 1 < NSCH)
                    def _(j=j, s=s):
                        fire(j + 1, s ^ 1)

                    drain_add(j, s)
                return carry

            lax.fori_loop(0, NSCH // 2, body, 0)
            plsc.subcore_barrier()
            pltpu.sync_copy(acc.at[pl.ds(r0, RPT)],
                            out_hbm.at[cid, pl.ds(r0, RPT)])

        return sc_scatter

    return sc_gather, make_scatter(TW), make_scatter(H)


def _sc_gather(tbl, row2, col2, ea):
    return _sc_kernels()[0](tbl, row2, col2, ea)


def _sc_scatter2(s, row2, z):
    return _sc_kernels()[1](s, row2, z)


def _sc_scatter1(s, row2, z):
    return _sc_kernels()[2](s, row2, z)


_BN = 2000
_BE = 2048


def _init_body(x_ref, pos_ref, w_ref, b_ref, o_ref):
    h = jnp.dot(x_ref[...], w_ref[...], preferred_element_type=jnp.float32)
    h = h + b_ref[...]
    o_ref[...] = jnp.concatenate(
        [h, pos_ref[...], jnp.zeros((h.shape[0], TW - H - 3), jnp.float32)],
        axis=1)


def _init_call(x, pos, in_w, in_b):
    return pl.pallas_call(
        _init_body,
        grid=(N // _BN,),
        in_specs=[
            pl.BlockSpec((_BN, F), lambda i: (i, 0)),
            pl.BlockSpec((_BN, 3), lambda i: (i, 0)),
            pl.BlockSpec((F, H), lambda i: (0, 0)),
            pl.BlockSpec((1, H), lambda i: (0, 0)),
        ],
        out_specs=pl.BlockSpec((_BN, TW), lambda i: (i, 0)),
        out_shape=jax.ShapeDtypeStruct((N, TW), jnp.float32),
    )(x, pos, in_w, in_b)


def _edge_body(has_coord, g_ref,
               w1, b1, w2, b2, cw1, cb1, cw2, cb2, o_ref):
    g = g_ref[...]
    diff = g[:, H:H + 3] - g[:, TW + H:TW + H + 3]
    dist = jnp.sqrt(jnp.sum(diff * diff, axis=1, keepdims=True) + 1e-12)
    msg_in = jnp.concatenate(
        [g[:, :H], g[:, TW:TW + H], dist, g[:, 2 * TW:2 * TW + ED]], axis=1)
    m1 = jnp.dot(msg_in, w1[...], preferred_element_type=jnp.float32) + b1[...]
    m1 = jax.nn.silu(m1)
    m = jax.nn.silu(jnp.dot(m1, w2[...], preferred_element_type=jnp.float32) + b2[...])
    pad = jnp.zeros((m.shape[0], 128 - TW), jnp.float32)
    if has_coord:
        cw = jax.nn.silu(jnp.dot(m, cw1[...], preferred_element_type=jnp.float32) + cb1[...])
        cw = jnp.dot(cw, cw2[...], preferred_element_type=jnp.float32) + cb2[...]
        o_ref[...] = jnp.concatenate(
            [m, diff * cw, jnp.zeros((m.shape[0], TW - H - 3), jnp.float32),
             pad], axis=1)
    else:
        o_ref[...] = jnp.concatenate(
            [m, jnp.zeros((m.shape[0], TW - H, ), jnp.float32), pad], axis=1)


def _edge_call(g, w1, b1, w2, b2, cw1, cb1, cw2, cb2, has_coord):
    full = lambda shape: pl.BlockSpec(shape, lambda i: (0, 0))
    return pl.pallas_call(
        functools.partial(_edge_body, has_coord),
        grid=(E_PAD // _BE,),
        in_specs=[
            pl.BlockSpec((_BE, 128), lambda i: (i, 0)),
            full((2 * H + 1 + ED, H)),
            full((1, H)), full((H, H)), full((1, H)),
            full((H, H)), full((1, H)), full((H, 1)), full((1, 1)),
        ],
        out_specs=pl.BlockSpec((_BE, 128), lambda i: (i, 0)),
        out_shape=jax.ShapeDtypeStruct((E_PAD, 128), jnp.float32),
    )(g, w1, b1, w2, b2, cw1, cb1, cw2, cb2)


def _node_body(has_coord, t_ref, p0_ref, p1_ref, w1, b1, w2, b2, o_ref):
    t = t_ref[...]
    h = t[:, :H]
    p0 = p0_ref[...]
    p1 = p1_ref[...]
    agg = p0[:, :H] + p1[:, :H]
    hin = jnp.concatenate([h, agg], axis=1)
    hn = jax.nn.silu(
        jnp.dot(hin, w1[...], preferred_element_type=jnp.float32) + b1[...])
    hn = jnp.dot(hn, w2[...], preferred_element_type=jnp.float32) + b2[...]
    h_new = h + hn
    if has_coord:
        pos = t[:, H:H + 3] + p0[:, H:H + 3] + p1[:, H:H + 3]
        o_ref[...] = jnp.concatenate(
            [h_new, pos, jnp.zeros((h_new.shape[0], TW - H - 3), jnp.float32)],
            axis=1)
    else:
        o_ref[...] = h_new


def _node_call(t, p0, p1, w1, b1, w2, b2, has_coord):
    full = lambda shape: pl.BlockSpec(shape, lambda i: (0, 0))
    pw = TW if has_coord else H
    return pl.pallas_call(
        functools.partial(_node_body, has_coord),
        grid=(N // _BN,),
        in_specs=[
            pl.BlockSpec((_BN, TW), lambda i: (i, 0)),
            pl.BlockSpec((_BN, pw), lambda i: (i, 0)),
            pl.BlockSpec((_BN, pw), lambda i: (i, 0)),
            full((2 * H, H)), full((1, H)), full((H, H)), full((1, H)),
        ],
        out_specs=pl.BlockSpec((_BN, pw), lambda i: (i, 0)),
        out_shape=jax.ShapeDtypeStruct((N, pw), jnp.float32),
    )(t, p0, p1, w1, b1, w2, b2)


_BP = 256
N_POOL = 10240


def _pool_body(h_ref, ids_ref, w1_ref, b1_ref, w2_ref, b2_ref, o_ref,
               sums, counts, mx):
    i = pl.program_id(0)
    nsteps = pl.num_programs(0)

    @pl.when(i == 0)
    def _():
        sums[...] = jnp.zeros_like(sums)
        counts[...] = jnp.zeros_like(counts)
        mx[...] = jnp.full_like(mx, -jnp.inf)

    h = h_ref[...]
    ids = ids_ref[...]
    oh = (ids == lax.broadcasted_iota(jnp.int32, (_BP, G), 1)).astype(jnp.float32)
    sums[...] += lax.dot_general(oh, h, (((0,), (0,)), ((), ())),
                                 preferred_element_type=jnp.float32)
    counts[...] += lax.dot_general(oh, jnp.ones((_BP, 1), jnp.float32),
                                   (((0,), (0,)), ((), ())),
                                   preferred_element_type=jnp.float32)
    masked = jnp.where(oh[:, :, None] > 0.0, h[:, None, :], -jnp.inf)
    mx[...] = jnp.maximum(mx[...], jnp.max(masked, axis=0))

    @pl.when(i == nsteps - 1)
    def _():
        c = counts[...]
        mean = sums[...] / jnp.maximum(c, 1.0)
        mxx = jnp.where(c > 0.0, mx[...], 0.0)
        pooled = jnp.concatenate([mean, mxx], axis=1)
        y = jnp.maximum(
            jnp.dot(pooled, w1_ref[...], preferred_element_type=jnp.float32)
            + b1_ref[...], 0.0)
        o_ref[...] = jnp.dot(y, w2_ref[...], preferred_element_type=jnp.float32) + b2_ref[...]


def _pool_call(h, ids, w1, b1, w2, b2):
    full = lambda shape: pl.BlockSpec(shape, lambda i: (0, 0))
    return pl.pallas_call(
        _pool_body,
        grid=(N_POOL // _BP,),
        in_specs=[
            pl.BlockSpec((_BP, H), lambda i: (i, 0)),
            pl.BlockSpec((_BP, 1), lambda i: (i, 0)),
            full((2 * H, H)), full((1, H)), full((H, 2)), full((1, 2)),
        ],
        out_specs=pl.BlockSpec((G, 2), lambda i: (0, 0)),
        out_shape=jax.ShapeDtypeStruct((G, 2), jnp.float32),
        scratch_shapes=[
            pltpu.VMEM((G, H), jnp.float32),
            pltpu.VMEM((G, 1), jnp.float32),
            pltpu.VMEM((G, H), jnp.float32),
        ],
    )(h, ids, w1, b1, w2, b2)


def kernel(x, pos, edge_attr, params, edge_index, batch):
    row = edge_index[0]
    col = edge_index[1]
    epad = E_PAD - E
    row2 = jnp.concatenate([row, jnp.full((epad,), N, jnp.int32)]).reshape(-1, CH)
    col2 = jnp.concatenate([col, jnp.full((epad,), N, jnp.int32)]).reshape(-1, CH)
    ea_p = jnp.concatenate([edge_attr, jnp.zeros((epad, ED), jnp.float32)])
    z48 = jnp.zeros((N_PAD, TW), jnp.float32)
    z32 = jnp.zeros((N_PAD, H), jnp.float32)

    r2 = lambda b: b.reshape(1, -1)
    t = _init_call(x, pos, params['in_W'], r2(params['in_b']))

    for i in range(L):
        p = params['layers'][i]
        has_coord = i < L - 1
        tbl = jnp.concatenate([t, jnp.zeros((N_PAD - N, TW), jnp.float32)])
        g = _sc_gather(tbl, row2, col2, ea_p)
        if has_coord:
            cw1, cb1, cw2, cb2 = p['c_W1'], r2(p['c_b1']), p['c_W2'], r2(p['c_b2'])
        else:
            cw1 = jnp.zeros((H, H), jnp.float32)
            cb1 = jnp.zeros((1, H), jnp.float32)
            cw2 = jnp.zeros((H, 1), jnp.float32)
            cb2 = jnp.zeros((1, 1), jnp.float32)
        s = _edge_call(g,
                       p['m_W1'], r2(p['m_b1']), p['m_W2'], r2(p['m_b2']),
                       cw1, cb1, cw2, cb2, has_coord)
        if has_coord:
            ph = _sc_scatter2(s, row2, z48)
        else:
            ph = _sc_scatter1(s, row2, z32)
        t = _node_call(t, ph[0, :N], ph[1, :N],
                       p['n_W1'], r2(p['n_b1']),
                       p['n_W2'], r2(p['n_b2']), has_coord)

    h_pool = jnp.concatenate([t, jnp.zeros((N_POOL - N, H), jnp.float32)])
    ids = jnp.concatenate([batch, jnp.full((N_POOL - N,), G, jnp.int32)])
    return _pool_call(h_pool, ids.reshape(-1, 1),
                      params['cls_W1'], r2(params['cls_b1']),
                      params['cls_W2'], r2(params['cls_b2']))


# speedup vs baseline: 1.0597x; 1.0597x over previous
import functools

import jax
import jax.numpy as jnp
from jax import lax
from jax.experimental import pallas as pl
from jax.experimental.pallas import tpu as pltpu
from jax.experimental.pallas import tpu_sc as plsc

N, E, F, H, ED, L, G = 10000, 320000, 128, 32, 16, 4, 64

TW = 48
N_PAD = 10016
NW = 32
E_PAD = 327680
EPW = E_PAD // NW
CH = 128
SCH = 512
K = SCH // CH
NSCH = EPW // SCH
RPT = N_PAD // 16


@functools.lru_cache(maxsize=1)
def _sc_kernels():
    mesh = plsc.VectorSubcoreMesh(core_axis_name="c", subcore_axis_name="s")
    cparams = pltpu.CompilerParams(use_tc_tiling_on_sc=False)

    @functools.partial(
        pl.kernel,
        mesh=mesh,
        compiler_params=cparams,
        out_type=jax.ShapeDtypeStruct((E_PAD, 128), jnp.float32),
        scratch_types=[
            pltpu.VMEM((K, CH), jnp.int32),
            pltpu.VMEM((K, CH), jnp.int32),
            pltpu.VMEM((K, CH), jnp.int32),
            pltpu.VMEM((K, CH), jnp.int32),
            pltpu.VMEM((SCH, TW), jnp.float32),
            pltpu.VMEM((SCH, TW), jnp.float32),
            pltpu.VMEM((SCH, TW), jnp.float32),
            pltpu.VMEM((SCH, TW), jnp.float32),
            pltpu.SemaphoreType.DMA,
            pltpu.SemaphoreType.DMA,
        ],
    )
    def sc_gather(tbl, row2, col2, g_out,
                  ir0, ir1, ic0, ic1, br0, br1, bc0, bc1, sem0, sem1):
        cid = lax.axis_index("c")
        sid = lax.axis_index("s")
        wid = sid * 2 + cid
        b0 = wid * EPW
        idx_r, idx_c = (ir0, ir1), (ic0, ic1)
        buf_r, buf_c = (br0, br1), (bc0, bc1)
        sems = (sem0, sem1)

        def fire(j, s):
            base = b0 + j * SCH
            pltpu.sync_copy(row2.at[pl.ds(base // CH, K)], idx_r[s])
            pltpu.sync_copy(col2.at[pl.ds(base // CH, K)], idx_c[s])
            for k in range(K):
                pltpu.async_copy(tbl.at[idx_r[s].at[k]],
                                 buf_r[s].at[pl.ds(k * CH, CH)], sems[s])
                pltpu.async_copy(tbl.at[idx_c[s].at[k]],
                                 buf_c[s].at[pl.ds(k * CH, CH)], sems[s])

        def drain_write(j, s):
            base = b0 + j * SCH
            dst_r = g_out.at[pl.ds(base, SCH), pl.ds(0, TW)]
            dst_c = g_out.at[pl.ds(base, SCH), pl.ds(TW, TW)]
            pltpu.make_async_copy(dst_r, buf_r[s], sems[s]).wait()
            pltpu.make_async_copy(dst_c, buf_c[s], sems[s]).wait()
            pltpu.sync_copy(buf_r[s], dst_r)
            pltpu.sync_copy(buf_c[s], dst_c)

        fire(0, 0)

        def body(jj, carry):
            for s in (0, 1):
                j = 2 * jj + s

                @pl.when(j + 1 < NSCH)
                def _(j=j, s=s):
                    fire(j + 1, s ^ 1)

                drain_write(j, s)
            return carry

        lax.fori_loop(0, NSCH // 2, body, 0)

    def make_scatter(width):
        @functools.partial(
            pl.kernel,
            mesh=mesh,
            compiler_params=cparams,
            out_type=jax.ShapeDtypeStruct((2, N_PAD, width), jnp.float32),
            scratch_types=[
                pltpu.VMEM((K, CH), jnp.int32),
                pltpu.VMEM((K, CH), jnp.int32),
                pltpu.VMEM((SCH, width), jnp.float32),
                pltpu.VMEM((SCH, width), jnp.float32),
                pltpu.VMEM_SHARED((N_PAD, width), jnp.float32),
                pltpu.SemaphoreType.DMA,
                pltpu.SemaphoreType.DMA,
            ],
        )
        def sc_scatter(s_hbm, row2, z_hbm, out_hbm,
                       i0, i1, bf0, bf1, acc, sem0, sem1):
            cid = lax.axis_index("c")
            sid = lax.axis_index("s")
            r0 = sid * RPT
            b0 = (cid * 16 + sid) * EPW
            idx_v = (i0, i1)
            buf = (bf0, bf1)
            sems = (sem0, sem1)

            def fire(j, s):
                base = b0 + j * SCH
                pltpu.async_copy(row2.at[pl.ds(base // CH, K)], idx_v[s], sems[s])
                pltpu.async_copy(s_hbm.at[pl.ds(base, SCH), pl.ds(0, width)],
                                 buf[s], sems[s])

            def drain_add(j, s):
                base = b0 + j * SCH
                pltpu.make_async_copy(row2.at[pl.ds(base // CH, K)],
                                      idx_v[s], sems[s]).wait()
                pltpu.make_async_copy(s_hbm.at[pl.ds(base, SCH), pl.ds(0, width)],
                                      buf[s], sems[s]).wait()
                for k in range(K):
                    pltpu.sync_copy(buf[s].at[pl.ds(k * CH, CH)],
                                    acc.at[idx_v[s].at[k]], add=True)

            fire(0, 0)
            pltpu.sync_copy(z_hbm.at[pl.ds(r0, RPT)], acc.at[pl.ds(r0, RPT)])
            plsc.subcore_barrier()

            def body(jj, carry):
                for s in (0, 1):
                    j = 2 * jj + s

                    @pl.when(j + 1 < NSCH)
                    def _(j=j, s=s):
                        fire(j + 1, s ^ 1)

                    drain_add(j, s)
                return carry

            lax.fori_loop(0, NSCH // 2, body, 0)
            plsc.subcore_barrier()
            pltpu.sync_copy(acc.at[pl.ds(r0, RPT)],
                            out_hbm.at[cid, pl.ds(r0, RPT)])

        return sc_scatter

    return sc_gather, make_scatter(TW), make_scatter(H)


def _sc_gather(tbl, row2, col2):
    return _sc_kernels()[0](tbl, row2, col2)


def _sc_scatter2(s, row2, z):
    return _sc_kernels()[1](s, row2, z)


def _sc_scatter1(s, row2, z):
    return _sc_kernels()[2](s, row2, z)


_BN = 2000
_BE = 4096


def _init_body(x_ref, pos_ref, w_ref, b_ref, o_ref):
    h = jnp.dot(x_ref[...], w_ref[...], preferred_element_type=jnp.float32)
    h = h + b_ref[...]
    o_ref[...] = jnp.concatenate(
        [h, pos_ref[...], jnp.zeros((h.shape[0], TW - H - 3), jnp.float32)],
        axis=1)


def _init_call(x, pos, in_w, in_b):
    return pl.pallas_call(
        _init_body,
        grid=(N // _BN,),
        in_specs=[
            pl.BlockSpec((_BN, F), lambda i: (i, 0)),
            pl.BlockSpec((_BN, 3), lambda i: (i, 0)),
            pl.BlockSpec((F, H), lambda i: (0, 0)),
            pl.BlockSpec((1, H), lambda i: (0, 0)),
        ],
        out_specs=pl.BlockSpec((_BN, TW), lambda i: (i, 0)),
        out_shape=jax.ShapeDtypeStruct((N, TW), jnp.float32),
    )(x, pos, in_w, in_b)


def _edge_body(has_coord, g_ref, ea_ref,
               w1, b1, w2, b2, cw1, cb1, cw2, cb2, o_ref):
    g = g_ref[...]
    diff = g[:, H:H + 3] - g[:, TW + H:TW + H + 3]
    dist = jnp.sqrt(jnp.sum(diff * diff, axis=1, keepdims=True) + 1e-12)
    msg_in = jnp.concatenate(
        [g[:, :H], g[:, TW:TW + H], dist, ea_ref[...]], axis=1)
    m1 = jnp.dot(msg_in, w1[...], preferred_element_type=jnp.float32) + b1[...]
    m1 = jax.nn.silu(m1)
    m = jax.nn.silu(jnp.dot(m1, w2[...], preferred_element_type=jnp.float32) + b2[...])
    pad = jnp.zeros((m.shape[0], 128 - TW), jnp.float32)
    if has_coord:
        cw = jax.nn.silu(jnp.dot(m, cw1[...], preferred_element_type=jnp.float32) + cb1[...])
        cw = jnp.dot(cw, cw2[...], preferred_element_type=jnp.float32) + cb2[...]
        o_ref[...] = jnp.concatenate(
            [m, diff * cw, jnp.zeros((m.shape[0], TW - H - 3), jnp.float32),
             pad], axis=1)
    else:
        o_ref[...] = jnp.concatenate(
            [m, jnp.zeros((m.shape[0], TW - H, ), jnp.float32), pad], axis=1)


def _edge_call(g, ea, w1, b1, w2, b2, cw1, cb1, cw2, cb2, has_coord):
    full = lambda shape: pl.BlockSpec(shape, lambda i: (0, 0))
    return pl.pallas_call(
        functools.partial(_edge_body, has_coord),
        grid=(E_PAD // _BE,),
        in_specs=[
            pl.BlockSpec((_BE, 128), lambda i: (i, 0)),
            pl.BlockSpec((_BE, ED), lambda i: (i, 0)),
            full((2 * H + 1 + ED, H)),
            full((1, H)), full((H, H)), full((1, H)),
            full((H, H)), full((1, H)), full((H, 1)), full((1, 1)),
        ],
        out_specs=pl.BlockSpec((_BE, 128), lambda i: (i, 0)),
        out_shape=jax.ShapeDtypeStruct((E_PAD, 128), jnp.float32),
    )(g, ea, w1, b1, w2, b2, cw1, cb1, cw2, cb2)


def _node_body(has_coord, t_ref, p0_ref, p1_ref, w1, b1, w2, b2, o_ref):
    t = t_ref[...]
    h = t[:, :H]
    p0 = p0_ref[...]
    p1 = p1_ref[...]
    agg = p0[:, :H] + p1[:, :H]
    hin = jnp.concatenate([h, agg], axis=1)
    hn = jax.nn.silu(
        jnp.dot(hin, w1[...], preferred_element_type=jnp.float32) + b1[...])
    hn = jnp.dot(hn, w2[...], preferred_element_type=jnp.float32) + b2[...]
    h_new = h + hn
    if has_coord:
        pos = t[:, H:H + 3] + p0[:, H:H + 3] + p1[:, H:H + 3]
        o_ref[...] = jnp.concatenate(
            [h_new, pos, jnp.zeros((h_new.shape[0], TW - H - 3), jnp.float32)],
            axis=1)
    else:
        o_ref[...] = h_new


def _node_call(t, p0, p1, w1, b1, w2, b2, has_coord):
    full = lambda shape: pl.BlockSpec(shape, lambda i: (0, 0))
    pw = TW if has_coord else H
    return pl.pallas_call(
        functools.partial(_node_body, has_coord),
        grid=(N // _BN,),
        in_specs=[
            pl.BlockSpec((_BN, TW), lambda i: (i, 0)),
            pl.BlockSpec((_BN, pw), lambda i: (i, 0)),
            pl.BlockSpec((_BN, pw), lambda i: (i, 0)),
            full((2 * H, H)), full((1, H)), full((H, H)), full((1, H)),
        ],
        out_specs=pl.BlockSpec((_BN, pw), lambda i: (i, 0)),
        out_shape=jax.ShapeDtypeStruct((N, pw), jnp.float32),
    )(t, p0, p1, w1, b1, w2, b2)


_BP = 256
N_POOL = 10240


def _pool_body(h_ref, ids_ref, w1_ref, b1_ref, w2_ref, b2_ref, o_ref,
               sums, counts, mx):
    i = pl.program_id(0)
    nsteps = pl.num_programs(0)

    @pl.when(i == 0)
    def _():
        sums[...] = jnp.zeros_like(sums)
        counts[...] = jnp.zeros_like(counts)
        mx[...] = jnp.full_like(mx, -jnp.inf)

    h = h_ref[...]
    ids = ids_ref[...]
    oh = (ids == lax.broadcasted_iota(jnp.int32, (_BP, G), 1)).astype(jnp.float32)
    sums[...] += lax.dot_general(oh, h, (((0,), (0,)), ((), ())),
                                 preferred_element_type=jnp.float32)
    counts[...] += lax.dot_general(oh, jnp.ones((_BP, 1), jnp.float32),
                                   (((0,), (0,)), ((), ())),
                                   preferred_element_type=jnp.float32)
    masked = jnp.where(oh[:, :, None] > 0.0, h[:, None, :], -jnp.inf)
    mx[...] = jnp.maximum(mx[...], jnp.max(masked, axis=0))

    @pl.when(i == nsteps - 1)
    def _():
        c = counts[...]
        mean = sums[...] / jnp.maximum(c, 1.0)
        mxx = jnp.where(c > 0.0, mx[...], 0.0)
        pooled = jnp.concatenate([mean, mxx], axis=1)
        y = jnp.maximum(
            jnp.dot(pooled, w1_ref[...], preferred_element_type=jnp.float32)
            + b1_ref[...], 0.0)
        o_ref[...] = jnp.dot(y, w2_ref[...], preferred_element_type=jnp.float32) + b2_ref[...]


def _pool_call(h, ids, w1, b1, w2, b2):
    full = lambda shape: pl.BlockSpec(shape, lambda i: (0, 0))
    return pl.pallas_call(
        _pool_body,
        grid=(N_POOL // _BP,),
        in_specs=[
            pl.BlockSpec((_BP, H), lambda i: (i, 0)),
            pl.BlockSpec((_BP, 1), lambda i: (i, 0)),
            full((2 * H, H)), full((1, H)), full((H, 2)), full((1, 2)),
        ],
        out_specs=pl.BlockSpec((G, 2), lambda i: (0, 0)),
        out_shape=jax.ShapeDtypeStruct((G, 2), jnp.float32),
        scratch_shapes=[
            pltpu.VMEM((G, H), jnp.float32),
            pltpu.VMEM((G, 1), jnp.float32),
            pltpu.VMEM((G, H), jnp.float32),
        ],
    )(h, ids, w1, b1, w2, b2)


def kernel(x, pos, edge_attr, params, edge_index, batch):
    row = edge_index[0]
    col = edge_index[1]
    epad = E_PAD - E
    row2 = jnp.concatenate([row, jnp.full((epad,), N, jnp.int32)]).reshape(-1, CH)
    col2 = jnp.concatenate([col, jnp.full((epad,), N, jnp.int32)]).reshape(-1, CH)
    ea_p = jnp.concatenate([edge_attr, jnp.zeros((epad, ED), jnp.float32)])
    z48 = jnp.zeros((N_PAD, TW), jnp.float32)
    z32 = jnp.zeros((N_PAD, H), jnp.float32)

    r2 = lambda b: b.reshape(1, -1)
    t = _init_call(x, pos, params['in_W'], r2(params['in_b']))

    for i in range(L):
        p = params['layers'][i]
        has_coord = i < L - 1
        tbl = jnp.concatenate([t, jnp.zeros((N_PAD - N, TW), jnp.float32)])
        g = _sc_gather(tbl, row2, col2)
        if has_coord:
            cw1, cb1, cw2, cb2 = p['c_W1'], r2(p['c_b1']), p['c_W2'], r2(p['c_b2'])
        else:
            cw1 = jnp.zeros((H, H), jnp.float32)
            cb1 = jnp.zeros((1, H), jnp.float32)
            cw2 = jnp.zeros((H, 1), jnp.float32)
            cb2 = jnp.zeros((1, 1), jnp.float32)
        s = _edge_call(g, ea_p,
                       p['m_W1'], r2(p['m_b1']), p['m_W2'], r2(p['m_b2']),
                       cw1, cb1, cw2, cb2, has_coord)
        if has_coord:
            ph = _sc_scatter2(s, row2, z48)
        else:
            ph = _sc_scatter1(s, row2, z32)
        t = _node_call(t, ph[0, :N], ph[1, :N],
                       p['n_W1'], r2(p['n_b1']),
                       p['n_W2'], r2(p['n_b2']), has_coord)

    h_pool = jnp.concatenate([t, jnp.zeros((N_POOL - N, H), jnp.float32)])
    ids = jnp.concatenate([batch, jnp.full((N_POOL - N,), G, jnp.int32)])
    return _pool_call(h_pool, ids.reshape(-1, 1),
                      params['cls_W1'], r2(params['cls_b1']),
                      params['cls_W2'], r2(params['cls_b2']))


# speedup vs baseline: 1.1894x; 1.1224x over previous
import functools

import jax
import jax.numpy as jnp
from jax import lax
from jax.experimental import pallas as pl
from jax.experimental.pallas import tpu as pltpu
from jax.experimental.pallas import tpu_sc as plsc

N, E, F, H, ED, L, G = 10000, 320000, 128, 32, 16, 4, 64

TW = 48
N_PAD = 10016
NW = 32
E_PAD = 327680
E2 = E_PAD // 2
EPW = E2 // NW
CH = 128
SCH = 512
K = SCH // CH
NSCH = EPW // SCH
RPT = N_PAD // 16


@functools.lru_cache(maxsize=1)
def _sc_kernels():
    mesh = plsc.VectorSubcoreMesh(core_axis_name="c", subcore_axis_name="s")
    cparams = pltpu.CompilerParams(use_tc_tiling_on_sc=False)

    def make_gather(hb):
        @functools.partial(
            pl.kernel,
            mesh=mesh,
            compiler_params=cparams,
            out_type=jax.ShapeDtypeStruct((E2, 128), jnp.float32),
            scratch_types=[
                pltpu.VMEM((K, CH), jnp.int32),
                pltpu.VMEM((K, CH), jnp.int32),
                pltpu.VMEM((K, CH), jnp.int32),
                pltpu.VMEM((K, CH), jnp.int32),
                pltpu.VMEM((SCH, TW), jnp.float32),
                pltpu.VMEM((SCH, TW), jnp.float32),
                pltpu.VMEM((SCH, TW), jnp.float32),
                pltpu.VMEM((SCH, TW), jnp.float32),
                pltpu.SemaphoreType.DMA,
                pltpu.SemaphoreType.DMA,
            ],
        )
        def sc_gather(tbl, row2, col2, g_out,
                      ir0, ir1, ic0, ic1, br0, br1, bc0, bc1, sem0, sem1):
            cid = lax.axis_index("c")
            sid = lax.axis_index("s")
            wid = sid * 2 + cid
            b0 = wid * EPW
            idx_r, idx_c = (ir0, ir1), (ic0, ic1)
            buf_r, buf_c = (br0, br1), (bc0, bc1)
            sems = (sem0, sem1)

            def fire(j, s):
                gbase = hb * E2 + b0 + j * SCH
                pltpu.sync_copy(row2.at[pl.ds(gbase // CH, K)], idx_r[s])
                pltpu.sync_copy(col2.at[pl.ds(gbase // CH, K)], idx_c[s])
                for k in range(K):
                    pltpu.async_copy(tbl.at[idx_r[s].at[k]],
                                     buf_r[s].at[pl.ds(k * CH, CH)], sems[s])
                    pltpu.async_copy(tbl.at[idx_c[s].at[k]],
                                     buf_c[s].at[pl.ds(k * CH, CH)], sems[s])

            def drain_write(j, s):
                base = b0 + j * SCH
                dst_r = g_out.at[pl.ds(base, SCH), pl.ds(0, TW)]
                dst_c = g_out.at[pl.ds(base, SCH), pl.ds(TW, TW)]
                pltpu.make_async_copy(dst_r, buf_r[s], sems[s]).wait()
                pltpu.make_async_copy(dst_c, buf_c[s], sems[s]).wait()
                pltpu.sync_copy(buf_r[s], dst_r)
                pltpu.sync_copy(buf_c[s], dst_c)

            fire(0, 0)

            def body(jj, carry):
                for s in (0, 1):
                    j = 2 * jj + s

                    @pl.when(j + 1 < NSCH)
                    def _(j=j, s=s):
                        fire(j + 1, s ^ 1)

                    drain_write(j, s)
                return carry

            lax.fori_loop(0, NSCH // 2, body, 0)

        return sc_gather

    def make_scatter(width, hb):
        @functools.partial(
            pl.kernel,
            mesh=mesh,
            compiler_params=cparams,
            out_type=jax.ShapeDtypeStruct((2, N_PAD, width), jnp.float32),
            scratch_types=[
                pltpu.VMEM((K, CH), jnp.int32),
                pltpu.VMEM((K, CH), jnp.int32),
                pltpu.VMEM((SCH, width), jnp.float32),
                pltpu.VMEM((SCH, width), jnp.float32),
                pltpu.VMEM_SHARED((N_PAD, width), jnp.float32),
                pltpu.SemaphoreType.DMA,
                pltpu.SemaphoreType.DMA,
            ],
        )
        def sc_scatter(s_hbm, row2, z_hbm, out_hbm,
                       i0, i1, bf0, bf1, acc, sem0, sem1):
            cid = lax.axis_index("c")
            sid = lax.axis_index("s")
            r0 = sid * RPT
            b0 = (cid * 16 + sid) * EPW
            idx_v = (i0, i1)
            buf = (bf0, bf1)
            sems = (sem0, sem1)

            def fire(j, s):
                base = b0 + j * SCH
                gbase = hb * E2 + base
                pltpu.async_copy(row2.at[pl.ds(gbase // CH, K)], idx_v[s], sems[s])
                pltpu.async_copy(s_hbm.at[pl.ds(base, SCH), pl.ds(0, width)],
                                 buf[s], sems[s])

            def drain_add(j, s):
                base = b0 + j * SCH
                gbase = hb * E2 + base
                pltpu.make_async_copy(row2.at[pl.ds(gbase // CH, K)],
                                      idx_v[s], sems[s]).wait()
                pltpu.make_async_copy(s_hbm.at[pl.ds(base, SCH), pl.ds(0, width)],
                                      buf[s], sems[s]).wait()
                for k in range(K):
                    pltpu.sync_copy(buf[s].at[pl.ds(k * CH, CH)],
                                    acc.at[idx_v[s].at[k]], add=True)

            fire(0, 0)
            pltpu.sync_copy(z_hbm.at[pl.ds(r0, RPT)], acc.at[pl.ds(r0, RPT)])
            plsc.subcore_barrier()

            def body(jj, carry):
                for s in (0, 1):
                    j = 2 * jj + s

                    @pl.when(j + 1 < NSCH)
                    def _(j=j, s=s):
                        fire(j + 1, s ^ 1)

                    drain_add(j, s)
                return carry

            lax.fori_loop(0, NSCH // 2, body, 0)
            plsc.subcore_barrier()
            pltpu.sync_copy(acc.at[pl.ds(r0, RPT)],
                            out_hbm.at[cid, pl.ds(r0, RPT)])

        return sc_scatter

    return {
        'g': (make_gather(0), make_gather(1)),
        's2': (make_scatter(TW, 0), make_scatter(TW, 1)),
        's1': (make_scatter(H, 0), make_scatter(H, 1)),
    }


def _sc_gather(hb, tbl, row2, col2):
    return _sc_kernels()['g'][hb](tbl, row2, col2)


def _sc_scatter2(hb, s, row2, z):
    return _sc_kernels()['s2'][hb](s, row2, z)


def _sc_scatter1(hb, s, row2, z):
    return _sc_kernels()['s1'][hb](s, row2, z)


_BN = 2000
_BE = 4096


def _init_body(x_ref, pos_ref, w_ref, b_ref, o_ref):
    h = jnp.dot(x_ref[...], w_ref[...], preferred_element_type=jnp.float32)
    h = h + b_ref[...]
    o_ref[...] = jnp.concatenate(
        [h, pos_ref[...], jnp.zeros((h.shape[0], TW - H - 3), jnp.float32)],
        axis=1)


def _init_call(x, pos, in_w, in_b):
    return pl.pallas_call(
        _init_body,
        grid=(N // _BN,),
        in_specs=[
            pl.BlockSpec((_BN, F), lambda i: (i, 0)),
            pl.BlockSpec((_BN, 3), lambda i: (i, 0)),
            pl.BlockSpec((F, H), lambda i: (0, 0)),
            pl.BlockSpec((1, H), lambda i: (0, 0)),
        ],
        out_specs=pl.BlockSpec((_BN, TW), lambda i: (i, 0)),
        out_shape=jax.ShapeDtypeStruct((N, TW), jnp.float32),
    )(x, pos, in_w, in_b)


def _edge_body(has_coord, g_ref, ea_ref,
               w1, b1, w2, b2, cw1, cb1, cw2, cb2, o_ref):
    g = g_ref[...]
    diff = g[:, H:H + 3] - g[:, TW + H:TW + H + 3]
    dist = jnp.sqrt(jnp.sum(diff * diff, axis=1, keepdims=True) + 1e-12)
    msg_in = jnp.concatenate(
        [g[:, :H], g[:, TW:TW + H], dist, ea_ref[...]], axis=1)
    m1 = jnp.dot(msg_in, w1[...], preferred_element_type=jnp.float32) + b1[...]
    m1 = jax.nn.silu(m1)
    m = jax.nn.silu(jnp.dot(m1, w2[...], preferred_element_type=jnp.float32) + b2[...])
    pad = jnp.zeros((m.shape[0], 128 - TW), jnp.float32)
    if has_coord:
        cw = jax.nn.silu(jnp.dot(m, cw1[...], preferred_element_type=jnp.float32) + cb1[...])
        cw = jnp.dot(cw, cw2[...], preferred_element_type=jnp.float32) + cb2[...]
        o_ref[...] = jnp.concatenate(
            [m, diff * cw, jnp.zeros((m.shape[0], TW - H - 3), jnp.float32),
             pad], axis=1)
    else:
        o_ref[...] = jnp.concatenate(
            [m, jnp.zeros((m.shape[0], TW - H, ), jnp.float32), pad], axis=1)


def _edge_call(g, ea, w1, b1, w2, b2, cw1, cb1, cw2, cb2, has_coord):
    full = lambda shape: pl.BlockSpec(shape, lambda i: (0, 0))
    return pl.pallas_call(
        functools.partial(_edge_body, has_coord),
        grid=(E2 // _BE,),
        in_specs=[
            pl.BlockSpec((_BE, 128), lambda i: (i, 0)),
            pl.BlockSpec((_BE, ED), lambda i: (i, 0)),
            full((2 * H + 1 + ED, H)),
            full((1, H)), full((H, H)), full((1, H)),
            full((H, H)), full((1, H)), full((H, 1)), full((1, 1)),
        ],
        out_specs=pl.BlockSpec((_BE, 128), lambda i: (i, 0)),
        out_shape=jax.ShapeDtypeStruct((E2, 128), jnp.float32),
    )(g, ea, w1, b1, w2, b2, cw1, cb1, cw2, cb2)


def _node_body(has_coord, t_ref, p0_ref, p1_ref, p2_ref, p3_ref,
               w1, b1, w2, b2, o_ref):
    t = t_ref[...]
    h = t[:, :H]
    p0 = p0_ref[...]
    p1 = p1_ref[...]
    p2 = p2_ref[...]
    p3 = p3_ref[...]
    agg = (p0[:, :H] + p1[:, :H]) + (p2[:, :H] + p3[:, :H])
    hin = jnp.concatenate([h, agg], axis=1)
    hn = jax.nn.silu(
        jnp.dot(hin, w1[...], preferred_element_type=jnp.float32) + b1[...])
    hn = jnp.dot(hn, w2[...], preferred_element_type=jnp.float32) + b2[...]
    h_new = h + hn
    if has_coord:
        pos = (t[:, H:H + 3] + (p0[:, H:H + 3] + p1[:, H:H + 3])
               + (p2[:, H:H + 3] + p3[:, H:H + 3]))
        o_ref[...] = jnp.concatenate(
            [h_new, pos, jnp.zeros((h_new.shape[0], TW - H - 3), jnp.float32)],
            axis=1)
    else:
        o_ref[...] = h_new


def _node_call(t, p0, p1, p2, p3, w1, b1, w2, b2, has_coord):
    full = lambda shape: pl.BlockSpec(shape, lambda i: (0, 0))
    pw = TW if has_coord else H
    pblk = pl.BlockSpec((_BN, pw), lambda i: (i, 0))
    return pl.pallas_call(
        functools.partial(_node_body, has_coord),
        grid=(N // _BN,),
        in_specs=[
            pl.BlockSpec((_BN, TW), lambda i: (i, 0)),
            pblk, pblk, pblk, pblk,
            full((2 * H, H)), full((1, H)), full((H, H)), full((1, H)),
        ],
        out_specs=pblk,
        out_shape=jax.ShapeDtypeStruct((N, pw), jnp.float32),
    )(t, p0, p1, p2, p3, w1, b1, w2, b2)


_BP = 256
N_POOL = 10240


def _pool_body(h_ref, ids_ref, w1_ref, b1_ref, w2_ref, b2_ref, o_ref,
               sums, counts, mx):
    i = pl.program_id(0)
    nsteps = pl.num_programs(0)

    @pl.when(i == 0)
    def _():
        sums[...] = jnp.zeros_like(sums)
        counts[...] = jnp.zeros_like(counts)
        mx[...] = jnp.full_like(mx, -jnp.inf)

    h = h_ref[...]
    ids = ids_ref[...]
    oh = (ids == lax.broadcasted_iota(jnp.int32, (_BP, G), 1)).astype(jnp.float32)
    sums[...] += lax.dot_general(oh, h, (((0,), (0,)), ((), ())),
                                 preferred_element_type=jnp.float32)
    counts[...] += lax.dot_general(oh, jnp.ones((_BP, 1), jnp.float32),
                                   (((0,), (0,)), ((), ())),
                                   preferred_element_type=jnp.float32)
    masked = jnp.where(oh[:, :, None] > 0.0, h[:, None, :], -jnp.inf)
    mx[...] = jnp.maximum(mx[...], jnp.max(masked, axis=0))

    @pl.when(i == nsteps - 1)
    def _():
        c = counts[...]
        mean = sums[...] / jnp.maximum(c, 1.0)
        mxx = jnp.where(c > 0.0, mx[...], 0.0)
        pooled = jnp.concatenate([mean, mxx], axis=1)
        y = jnp.maximum(
            jnp.dot(pooled, w1_ref[...], preferred_element_type=jnp.float32)
            + b1_ref[...], 0.0)
        o_ref[...] = jnp.dot(y, w2_ref[...], preferred_element_type=jnp.float32) + b2_ref[...]


def _pool_call(h, ids, w1, b1, w2, b2):
    full = lambda shape: pl.BlockSpec(shape, lambda i: (0, 0))
    return pl.pallas_call(
        _pool_body,
        grid=(N_POOL // _BP,),
        in_specs=[
            pl.BlockSpec((_BP, H), lambda i: (i, 0)),
            pl.BlockSpec((_BP, 1), lambda i: (i, 0)),
            full((2 * H, H)), full((1, H)), full((H, 2)), full((1, 2)),
        ],
        out_specs=pl.BlockSpec((G, 2), lambda i: (0, 0)),
        out_shape=jax.ShapeDtypeStruct((G, 2), jnp.float32),
        scratch_shapes=[
            pltpu.VMEM((G, H), jnp.float32),
            pltpu.VMEM((G, 1), jnp.float32),
            pltpu.VMEM((G, H), jnp.float32),
        ],
    )(h, ids, w1, b1, w2, b2)


def kernel(x, pos, edge_attr, params, edge_index, batch):
    row = edge_index[0]
    col = edge_index[1]
    epad = E_PAD - E
    row2 = jnp.concatenate([row, jnp.full((epad,), N, jnp.int32)]).reshape(-1, CH)
    col2 = jnp.concatenate([col, jnp.full((epad,), N, jnp.int32)]).reshape(-1, CH)
    ea_p = jnp.concatenate([edge_attr, jnp.zeros((epad, ED), jnp.float32)])
    z48 = jnp.zeros((N_PAD, TW), jnp.float32)
    z32 = jnp.zeros((N_PAD, H), jnp.float32)

    r2 = lambda b: b.reshape(1, -1)
    t = _init_call(x, pos, params['in_W'], r2(params['in_b']))

    for i in range(L):
        p = params['layers'][i]
        has_coord = i < L - 1
        tbl = jnp.concatenate([t, jnp.zeros((N_PAD - N, TW), jnp.float32)])
        ga = _sc_gather(0, tbl, row2, col2)
        gb = _sc_gather(1, tbl, row2, col2)
        if has_coord:
            cw1, cb1, cw2, cb2 = p['c_W1'], r2(p['c_b1']), p['c_W2'], r2(p['c_b2'])
        else:
            cw1 = jnp.zeros((H, H), jnp.float32)
            cb1 = jnp.zeros((1, H), jnp.float32)
            cw2 = jnp.zeros((H, 1), jnp.float32)
            cb2 = jnp.zeros((1, 1), jnp.float32)
        ew = (p['m_W1'], r2(p['m_b1']), p['m_W2'], r2(p['m_b2']),
              cw1, cb1, cw2, cb2, has_coord)
        sa = _edge_call(ga, ea_p[:E2], *ew)
        sb = _edge_call(gb, ea_p[E2:], *ew)
        if has_coord:
            pa = _sc_scatter2(0, sa, row2, z48)
            pb = _sc_scatter2(1, sb, row2, z48)
        else:
            pa = _sc_scatter1(0, sa, row2, z32)
            pb = _sc_scatter1(1, sb, row2, z32)
        t = _node_call(t, pa[0, :N], pa[1, :N], pb[0, :N], pb[1, :N],
                       p['n_W1'], r2(p['n_b1']),
                       p['n_W2'], r2(p['n_b2']), has_coord)

    h_pool = jnp.concatenate([t, jnp.zeros((N_POOL - N, H), jnp.float32)])
    ids = jnp.concatenate([batch, jnp.full((N_POOL - N,), G, jnp.int32)])
    return _pool_call(h_pool, ids.reshape(-1, 1),
                      params['cls_W1'], r2(params['cls_b1']),
                      params['cls_W2'], r2(params['cls_b2']))
